# baseline (device time: 68783 ns/iter reference)
import jax
import jax.numpy as jnp
from jax import lax
from jax.experimental import pallas as pl
from jax.experimental.pallas import tpu as pltpu


def kernel(x, dy):
    m, d = x.shape
    _, f = dy.shape
    blk = d // 4

    x16 = x.astype(jnp.bfloat16)
    dy16 = dy.astype(jnp.bfloat16)

    def body(x_ref, dy_ref, out_ref,
             xsend, xrecv, ysend, yrecv,
             xs_sem, xr_sem, ys_sem, yr_sem):
        my_x = lax.axis_index("x")
        my_y = lax.axis_index("y")
        b = 2 * my_x + my_y
        bx = 2 * (1 - my_x) + my_y

        barrier = pltpu.get_barrier_semaphore()
        for nbr in [(1 - my_x, my_y), (my_x, 1 - my_y)]:
            pl.semaphore_signal(
                barrier, inc=1,
                device_id=nbr, device_id_type=pl.DeviceIdType.MESH,
            )
        pl.semaphore_wait(barrier, 2)

        xb = x_ref[:, pl.ds(bx * blk, blk)]
        B = lax.dot_general(
            xb, dy_ref[:, :], (((0,), (0,)), ((), ())),
            preferred_element_type=jnp.float32,
        )
        xsend[:, :] = B.astype(jnp.bfloat16)

        rdma_x = pltpu.make_async_remote_copy(
            src_ref=xsend, dst_ref=xrecv,
            send_sem=xs_sem, recv_sem=xr_sem,
            device_id=(1 - my_x, my_y),
            device_id_type=pl.DeviceIdType.MESH,
        )
        rdma_x.start()

        xa = x_ref[:, pl.ds(b * blk, blk)]
        A = lax.dot_general(
            xa, dy_ref[:, :], (((0,), (0,)), ((), ())),
            preferred_element_type=jnp.float32,
        )

        rdma_x.wait()
        S = A + xrecv[:, :].astype(jnp.float32)
        out_ref[pl.ds(my_y * blk, blk), :] = S
        ysend[:, :] = S.astype(jnp.bfloat16)

        rdma_y = pltpu.make_async_remote_copy(
            src_ref=ysend, dst_ref=yrecv,
            send_sem=ys_sem, recv_sem=yr_sem,
            device_id=(my_x, 1 - my_y),
            device_id_type=pl.DeviceIdType.MESH,
        )
        rdma_y.start()
        rdma_y.wait()
        out_ref[pl.ds((1 - my_y) * blk, blk), :] = yrecv[:, :].astype(jnp.float32)

    return pl.pallas_call(
        body,
        out_shape=jax.ShapeDtypeStruct((d // 2, f), jnp.float32),
        in_specs=[
            pl.BlockSpec(memory_space=pltpu.VMEM),
            pl.BlockSpec(memory_space=pltpu.VMEM),
        ],
        out_specs=pl.BlockSpec(memory_space=pltpu.VMEM),
        scratch_shapes=[
            pltpu.VMEM((blk, f), jnp.bfloat16),
            pltpu.VMEM((blk, f), jnp.bfloat16),
            pltpu.VMEM((blk, f), jnp.bfloat16),
            pltpu.VMEM((blk, f), jnp.bfloat16),
            pltpu.SemaphoreType.DMA,
            pltpu.SemaphoreType.DMA,
            pltpu.SemaphoreType.DMA,
            pltpu.SemaphoreType.DMA,
        ],
        compiler_params=pltpu.CompilerParams(collective_id=0),
    )(x16, dy16)


# device time: 49840 ns/iter; 1.3801x vs baseline; 1.3801x over previous
import jax
import jax.numpy as jnp
from jax import lax
from jax.experimental import pallas as pl
from jax.experimental.pallas import tpu as pltpu

NC = 8


def kernel(x, dy):
    m, d = x.shape
    _, f = dy.shape
    blk = d // 4
    fc = f // NC

    def body(x_ref, dy_ref, out_ref,
             dy16, xsend, xrecv, ysend, yrecv,
             xs_sems, xr_sems, ys_sems, yr_sems):
        my_x = lax.axis_index("x")
        my_y = lax.axis_index("y")
        b = 2 * my_x + my_y
        bx = 2 * (1 - my_x) + my_y

        barrier = pltpu.get_barrier_semaphore()
        for nbr in [(1 - my_x, my_y), (my_x, 1 - my_y)]:
            pl.semaphore_signal(
                barrier, inc=1,
                device_id=nbr, device_id_type=pl.DeviceIdType.MESH,
            )
        pl.semaphore_wait(barrier, 2)

        dy16[:, :] = dy_ref[:, :].astype(jnp.bfloat16)
        xb = x_ref[:, pl.ds(bx * blk, blk)].astype(jnp.bfloat16)
        xa = x_ref[:, pl.ds(b * blk, blk)].astype(jnp.bfloat16)

        x_rdmas = []
        for c in range(NC):
            sl = pl.ds(c * fc, fc)
            xsend[:, sl] = lax.dot_general(
                xb, dy16[:, sl], (((0,), (0,)), ((), ())),
                preferred_element_type=jnp.float32,
            ).astype(jnp.bfloat16)
            rdma = pltpu.make_async_remote_copy(
                src_ref=xsend.at[:, sl], dst_ref=xrecv.at[:, sl],
                send_sem=xs_sems.at[c], recv_sem=xr_sems.at[c],
                device_id=(1 - my_x, my_y),
                device_id_type=pl.DeviceIdType.MESH,
            )
            rdma.start()
            x_rdmas.append(rdma)

        A = lax.dot_general(
            xa, dy16[:, :], (((0,), (0,)), ((), ())),
            preferred_element_type=jnp.float32,
        )

        y_rdmas = []
        for c in range(NC):
            sl = pl.ds(c * fc, fc)
            x_rdmas[c].wait_recv()
            S = A[:, c * fc:(c + 1) * fc] + xrecv[:, sl].astype(jnp.float32)
            out_ref[pl.ds(my_y * blk, blk), sl] = S
            ysend[:, sl] = S.astype(jnp.bfloat16)
            rdma = pltpu.make_async_remote_copy(
                src_ref=ysend.at[:, sl], dst_ref=yrecv.at[:, sl],
                send_sem=ys_sems.at[c], recv_sem=yr_sems.at[c],
                device_id=(my_x, 1 - my_y),
                device_id_type=pl.DeviceIdType.MESH,
            )
            rdma.start()
            y_rdmas.append(rdma)

        for c in range(NC):
            sl = pl.ds(c * fc, fc)
            y_rdmas[c].wait_recv()
            out_ref[pl.ds((1 - my_y) * blk, blk), sl] = (
                yrecv[:, sl].astype(jnp.float32)
            )
        for c in range(NC):
            x_rdmas[c].wait_send()
            y_rdmas[c].wait_send()

    return pl.pallas_call(
        body,
        out_shape=jax.ShapeDtypeStruct((d // 2, f), jnp.float32),
        in_specs=[
            pl.BlockSpec(memory_space=pltpu.VMEM),
            pl.BlockSpec(memory_space=pltpu.VMEM),
        ],
        out_specs=pl.BlockSpec(memory_space=pltpu.VMEM),
        scratch_shapes=[
            pltpu.VMEM((m, f), jnp.bfloat16),
            pltpu.VMEM((blk, f), jnp.bfloat16),
            pltpu.VMEM((blk, f), jnp.bfloat16),
            pltpu.VMEM((blk, f), jnp.bfloat16),
            pltpu.VMEM((blk, f), jnp.bfloat16),
            pltpu.SemaphoreType.DMA((NC,)),
            pltpu.SemaphoreType.DMA((NC,)),
            pltpu.SemaphoreType.DMA((NC,)),
            pltpu.SemaphoreType.DMA((NC,)),
        ],
        compiler_params=pltpu.CompilerParams(
            collective_id=0, vmem_limit_bytes=100 * 1024 * 1024,
        ),
    )(x, dy)
